# Initial kernel scaffold; baseline (speedup 1.0000x reference)
import functools

import jax
import jax.numpy as jnp
from jax import lax
from jax.experimental import pallas as pl
from jax.experimental.pallas import tpu as pltpu
from jax.experimental.pallas import tpu_sc as plsc

_NC, _NS = 2, 16
_NW = _NC * _NS


def _sc_gather(table, idx):
    n = idx.shape[0]
    e = table.shape[1]
    b_per_w = n // _NW
    assert n % _NW == 0 and b_per_w % 8 == 0

    mesh = plsc.VectorSubcoreMesh(core_axis_name="c", subcore_axis_name="s")

    @functools.partial(
        pl.kernel,
        mesh=mesh,
        out_type=jax.ShapeDtypeStruct((n, e), jnp.float32),
        scratch_types=[
            pltpu.VMEM((b_per_w,), jnp.int32),
            pltpu.VMEM((b_per_w, e), jnp.float32),
            pltpu.SemaphoreType.DMA,
        ],
    )
    def gather_k(idx_hbm, table_hbm, out_hbm, idx_v, rows_v, sem):
        wid = lax.axis_index("s") * _NC + lax.axis_index("c")
        base = wid * b_per_w
        pltpu.sync_copy(idx_hbm.at[pl.ds(base, b_per_w)], idx_v)
        pltpu.async_copy(table_hbm.at[idx_v], rows_v, sem).wait()
        pltpu.sync_copy(rows_v, out_hbm.at[pl.ds(base, b_per_w)])

    return gather_k(idx, table)


def _lstm_proj_body(feat_ref, emb_ref, wih_ref, whh_ref, bias_ref,
                    wout_ref, bout_ref, out_ref, *, seq_len, hidden, vocab):
    bb = feat_ref.shape[0]
    wih = wih_ref[...]
    whh = whh_ref[...]
    bias = bias_ref[...]
    wout = wout_ref[...]
    bout = bout_ref[...]
    h = jnp.zeros((bb, hidden), jnp.float32)
    c = jnp.zeros((bb, hidden), jnp.float32)
    for t in range(seq_len):
        xt = feat_ref[...] if t == 0 else emb_ref[:, t - 1, :]
        gates = (jnp.dot(xt, wih, preferred_element_type=jnp.float32)
                 + jnp.dot(h, whh, preferred_element_type=jnp.float32)
                 + bias)
        gi = jax.nn.sigmoid(gates[:, 0:hidden])
        gf = jax.nn.sigmoid(gates[:, hidden:2 * hidden])
        gg = jnp.tanh(gates[:, 2 * hidden:3 * hidden])
        go = jax.nn.sigmoid(gates[:, 3 * hidden:4 * hidden])
        c = gf * c + gi * gg
        h = go * jnp.tanh(c)
        logits = jnp.dot(h, wout, preferred_element_type=jnp.float32) + bout
        out_ref[:, t, :] = logits[:, :vocab]


def kernel(features, captions, embed_table, W_ih, W_hh, b_ih, b_hh, W_out, b_out):
    b, seq_len = captions.shape
    e = embed_table.shape[1]
    hidden = W_hh.shape[1]
    vocab = W_out.shape[0]
    vpad = (vocab + 127) // 128 * 128

    idx = captions[:, :seq_len - 1].reshape(-1).astype(jnp.int32)
    emb = _sc_gather(embed_table, idx).reshape(b, seq_len - 1, e)

    wih = W_ih.T
    whh = W_hh.T
    bias = (b_ih + b_hh)[None, :]
    wout = jnp.zeros((hidden, vpad), jnp.float32).at[:, :vocab].set(W_out.T)
    bout = jnp.zeros((1, vpad), jnp.float32).at[:, :vocab].set(b_out[None, :])

    bb = 256
    body = functools.partial(_lstm_proj_body, seq_len=seq_len, hidden=hidden,
                             vocab=vocab)
    out = pl.pallas_call(
        body,
        grid=(b // bb,),
        in_specs=[
            pl.BlockSpec((bb, e), lambda i: (i, 0)),
            pl.BlockSpec((bb, seq_len - 1, e), lambda i: (i, 0, 0)),
            pl.BlockSpec((e, 4 * hidden), lambda i: (0, 0)),
            pl.BlockSpec((hidden, 4 * hidden), lambda i: (0, 0)),
            pl.BlockSpec((1, 4 * hidden), lambda i: (0, 0)),
            pl.BlockSpec((hidden, vpad), lambda i: (0, 0)),
            pl.BlockSpec((1, vpad), lambda i: (0, 0)),
        ],
        out_specs=pl.BlockSpec((bb, seq_len, vocab), lambda i: (i, 0, 0)),
        out_shape=jax.ShapeDtypeStruct((b, seq_len, vocab), jnp.float32),
    )(features, emb, wih, whh, bias, wout, bout)
    return out


# speedup vs baseline: 1.1649x; 1.1649x over previous
import functools

import jax
import jax.numpy as jnp
from jax import lax
from jax.experimental import pallas as pl
from jax.experimental.pallas import tpu as pltpu
from jax.experimental.pallas import tpu_sc as plsc

_NC, _NS = 2, 16
_NW = _NC * _NS


def _sc_gather(table, idx):
    n = idx.shape[0]
    e = table.shape[1]
    b_per_w = n // _NW
    assert n % _NW == 0 and b_per_w % 8 == 0

    mesh = plsc.VectorSubcoreMesh(core_axis_name="c", subcore_axis_name="s")

    @functools.partial(
        pl.kernel,
        mesh=mesh,
        out_type=jax.ShapeDtypeStruct((n, e), jnp.float32),
        scratch_types=[
            pltpu.VMEM((b_per_w,), jnp.int32),
            pltpu.VMEM((b_per_w, e), jnp.float32),
            pltpu.SemaphoreType.DMA,
        ],
    )
    def gather_k(idx_hbm, table_hbm, out_hbm, idx_v, rows_v, sem):
        wid = lax.axis_index("s") * _NC + lax.axis_index("c")
        base = wid * b_per_w
        pltpu.sync_copy(idx_hbm.at[pl.ds(base, b_per_w)], idx_v)
        pltpu.async_copy(table_hbm.at[idx_v], rows_v, sem).wait()
        pltpu.sync_copy(rows_v, out_hbm.at[pl.ds(base, b_per_w)])

    return gather_k(idx, table)


def _lstm_proj_body(feat_ref, emb_ref, wih_ref, whh_ref, bias_ref,
                    wout_ref, bout_ref, out_ref, *, seq_len, hidden, vocab):
    bb = feat_ref.shape[0]
    wih = wih_ref[...]
    whh = whh_ref[...]
    bias = bias_ref[...]
    wout = wout_ref[...]
    bout = bout_ref[...]
    h = jnp.zeros((bb, hidden), jnp.float32)
    c = jnp.zeros((bb, hidden), jnp.float32)
    for t in range(seq_len):
        xt = feat_ref[...] if t == 0 else emb_ref[:, t - 1, :]
        gates = (jnp.dot(xt, wih, preferred_element_type=jnp.float32)
                 + jnp.dot(h, whh, preferred_element_type=jnp.float32)
                 + bias)
        gi = jax.nn.sigmoid(gates[:, 0:hidden])
        gf = jax.nn.sigmoid(gates[:, hidden:2 * hidden])
        gg = jnp.tanh(gates[:, 2 * hidden:3 * hidden])
        go = jax.nn.sigmoid(gates[:, 3 * hidden:4 * hidden])
        c = gf * c + gi * gg
        h = go * jnp.tanh(c)
        logits = jnp.dot(h, wout, preferred_element_type=jnp.float32) + bout
        out_ref[:, t, :] = logits[:, :vocab]


def kernel(features, captions, embed_table, W_ih, W_hh, b_ih, b_hh, W_out, b_out):
    b, seq_len = captions.shape
    e = embed_table.shape[1]
    hidden = W_hh.shape[1]
    vocab = W_out.shape[0]
    vpad = (vocab + 127) // 128 * 128

    idx = captions[:, :seq_len - 1].reshape(-1).astype(jnp.int32)
    emb = _sc_gather(embed_table, idx).reshape(b, seq_len - 1, e)

    wih = W_ih.T
    whh = W_hh.T
    bias = (b_ih + b_hh)[None, :]
    wout = jnp.zeros((hidden, vpad), jnp.float32).at[:, :vocab].set(W_out.T)
    bout = jnp.zeros((1, vpad), jnp.float32).at[:, :vocab].set(b_out[None, :])

    bb = 128
    body = functools.partial(_lstm_proj_body, seq_len=seq_len, hidden=hidden,
                             vocab=vocab)
    out = pl.pallas_call(
        body,
        grid=(b // bb,),
        in_specs=[
            pl.BlockSpec((bb, e), lambda i: (i, 0)),
            pl.BlockSpec((bb, seq_len - 1, e), lambda i: (i, 0, 0)),
            pl.BlockSpec((e, 4 * hidden), lambda i: (0, 0)),
            pl.BlockSpec((hidden, 4 * hidden), lambda i: (0, 0)),
            pl.BlockSpec((1, 4 * hidden), lambda i: (0, 0)),
            pl.BlockSpec((hidden, vpad), lambda i: (0, 0)),
            pl.BlockSpec((1, vpad), lambda i: (0, 0)),
        ],
        out_specs=pl.BlockSpec((bb, seq_len, vocab), lambda i: (i, 0, 0)),
        out_shape=jax.ShapeDtypeStruct((b, seq_len, vocab), jnp.float32),
    )(features, emb, wih, whh, bias, wout, bout)
    return out


# speedup vs baseline: 2.8032x; 2.4065x over previous
import functools

import jax
import jax.numpy as jnp
from jax import lax
from jax.experimental import pallas as pl
from jax.experimental.pallas import tpu as pltpu
from jax.experimental.pallas import tpu_sc as plsc

_NC, _NS = 2, 16
_NW = _NC * _NS


def _sc_gather(table, idx):
    n = idx.shape[0]
    e = table.shape[1]
    b_per_w = n // _NW
    assert n % _NW == 0 and b_per_w % 8 == 0

    mesh = plsc.VectorSubcoreMesh(core_axis_name="c", subcore_axis_name="s")

    @functools.partial(
        pl.kernel,
        mesh=mesh,
        out_type=jax.ShapeDtypeStruct((n, e), jnp.float32),
        scratch_types=[
            pltpu.VMEM((b_per_w,), jnp.int32),
            pltpu.VMEM((b_per_w, e), jnp.float32),
            pltpu.SemaphoreType.DMA,
        ],
    )
    def gather_k(idx_hbm, table_hbm, out_hbm, idx_v, rows_v, sem):
        wid = lax.axis_index("s") * _NC + lax.axis_index("c")
        base = wid * b_per_w
        pltpu.sync_copy(idx_hbm.at[pl.ds(base, b_per_w)], idx_v)
        pltpu.async_copy(table_hbm.at[idx_v], rows_v, sem).wait()
        pltpu.sync_copy(rows_v, out_hbm.at[pl.ds(base, b_per_w)])

    return gather_k(idx, table)


def _lstm_proj_body(feat_ref, emb_ref, wih_ref, whh_ref, bias_ref,
                    wout_ref, bout_ref, out_ref, ht_ref, ct_ref, *, hidden):
    t = pl.program_id(0)

    @pl.when(t == 0)
    def _init():
        ht_ref[...] = jnp.zeros_like(ht_ref)
        ct_ref[...] = jnp.zeros_like(ct_ref)

    xt = jnp.where(t == 0, feat_ref[...], emb_ref[0])
    xt_t = xt.T
    ht = ht_ref[...]
    gates = (jnp.dot(wih_ref[...], xt_t, preferred_element_type=jnp.float32)
             + jnp.dot(whh_ref[...], ht, preferred_element_type=jnp.float32)
             + bias_ref[...])
    gi = jax.nn.sigmoid(gates[0:hidden])
    gf = jax.nn.sigmoid(gates[hidden:2 * hidden])
    gg = jnp.tanh(gates[2 * hidden:3 * hidden])
    go = jax.nn.sigmoid(gates[3 * hidden:4 * hidden])
    ct = gf * ct_ref[...] + gi * gg
    ht = go * jnp.tanh(ct)
    ct_ref[...] = ct
    ht_ref[...] = ht
    out_ref[0] = (jnp.dot(wout_ref[...], ht, preferred_element_type=jnp.float32)
                  + bout_ref[...])


def kernel(features, captions, embed_table, W_ih, W_hh, b_ih, b_hh, W_out, b_out):
    b, seq_len = captions.shape
    e = embed_table.shape[1]
    hidden = W_hh.shape[1]
    vocab = W_out.shape[0]

    idx = captions[:, :seq_len - 1].T.reshape(-1).astype(jnp.int32)
    emb = _sc_gather(embed_table, idx).reshape(seq_len - 1, b, e)

    bias = (b_ih + b_hh).reshape(4 * hidden, 1)
    bout = b_out.reshape(vocab, 1)

    body = functools.partial(_lstm_proj_body, hidden=hidden)
    out_t = pl.pallas_call(
        body,
        grid=(seq_len,),
        in_specs=[
            pl.BlockSpec((b, e), lambda t: (0, 0)),
            pl.BlockSpec((1, b, e), lambda t: (jnp.maximum(t - 1, 0), 0, 0)),
            pl.BlockSpec((4 * hidden, e), lambda t: (0, 0)),
            pl.BlockSpec((4 * hidden, hidden), lambda t: (0, 0)),
            pl.BlockSpec((4 * hidden, 1), lambda t: (0, 0)),
            pl.BlockSpec((vocab, hidden), lambda t: (0, 0)),
            pl.BlockSpec((vocab, 1), lambda t: (0, 0)),
        ],
        out_specs=pl.BlockSpec((1, vocab, b), lambda t: (t, 0, 0)),
        out_shape=jax.ShapeDtypeStruct((seq_len, vocab, b), jnp.float32),
        scratch_shapes=[
            pltpu.VMEM((hidden, b), jnp.float32),
            pltpu.VMEM((hidden, b), jnp.float32),
        ],
    )(features, emb, W_ih, W_hh, bias, W_out, bout)
    return out_t.transpose(2, 0, 1)
